# baseline (speedup 1.0000x reference)
import functools
import jax
import jax.numpy as jnp
from jax import lax
from jax.experimental import pallas as pl
from jax.experimental.pallas import tpu as pltpu
from jax.experimental.pallas import tpu_sc as plsc

NC = 2
NS = 16
NW = NC * NS
C = 80
NK = 5
NB = 4
LK = 3


@functools.lru_cache(maxsize=None)
def _build(N, E, D):
    EW = E // NW
    NCHUNK = EW // C
    assert E % NW == 0 and EW % C == 0

    mesh = plsc.VectorSubcoreMesh(core_axis_name="c", subcore_axis_name="s")

    @functools.partial(
        pl.kernel,
        out_type=jax.ShapeDtypeStruct((NC, 1, N), jnp.float32),
        mesh=mesh,
        scratch_types=[
            pltpu.VMEM((NCHUNK, C), jnp.int32),
            pltpu.VMEM((C,), jnp.float32),
            pltpu.VMEM_SHARED((N,), jnp.float32),
            pltpu.SemaphoreType.DMA,
        ],
    )
    def deg_kernel(col3_hbm, zeros1_hbm, out_hbm, cidx, ones_v, acc_sh, sem):
        cid = lax.axis_index("c")
        sid = lax.axis_index("s")
        wid = sid * NC + cid
        for j in range(C // 16):
            ones_v[pl.ds(16 * j, 16)] = jnp.full((16,), 1.0, jnp.float32)

        @pl.when(sid == 0)
        def _():
            pltpu.sync_copy(zeros1_hbm, acc_sh)

        pltpu.sync_copy(col3_hbm.at[wid], cidx)
        plsc.subcore_barrier()

        def chunk(i, carry):
            @pl.when(i >= 8)
            def _():
                pltpu.make_async_copy(ones_v, acc_sh.at[cidx.at[0]], sem).wait()

            pltpu.async_copy(ones_v, acc_sh.at[cidx.at[i]], sem, add=True)
            return carry

        lax.fori_loop(0, NCHUNK, chunk, 0)
        for _ in range(8):
            pltpu.make_async_copy(ones_v, acc_sh.at[cidx.at[0]], sem).wait()
        plsc.subcore_barrier()

        @pl.when(sid == 0)
        def _():
            pltpu.sync_copy(acc_sh, out_hbm.at[cid, 0])

    NG = NCHUNK // NK
    assert NCHUNK % NK == 0

    @functools.partial(
        pl.kernel,
        out_type=jax.ShapeDtypeStruct((NC, N, D), jnp.float32),
        mesh=mesh,
        scratch_types=[
            pltpu.VMEM((2, NK, C), jnp.int32),
            pltpu.VMEM((2, NK, C), jnp.int32),
            pltpu.VMEM((NB, C, D), jnp.float32),
            pltpu.VMEM_SHARED((N, D), jnp.float32),
            pltpu.SemaphoreType.DMA((NB,)),
            pltpu.SemaphoreType.DMA((NB,)),
            pltpu.SemaphoreType.DMA,
        ],
    )
    def edge_kernel(z_hbm, row4_hbm, col4_hbm, zeros2_hbm, out_hbm,
                    ridx, cidx, rows_v, acc_sh, gsem, ssem, isem):
        cid = lax.axis_index("c")
        sid = lax.axis_index("s")
        wid = sid * NC + cid

        pltpu.sync_copy(row4_hbm.at[wid, 0], ridx.at[0])
        pltpu.sync_copy(col4_hbm.at[wid, 0], cidx.at[0])
        for b in range(LK):
            pltpu.async_copy(z_hbm.at[ridx.at[0, b]], rows_v.at[b], gsem.at[b])

        off = pl.multiple_of(sid * 640, 8)

        @pl.when((sid < 15) & (cid == 0))
        def _():
            pltpu.sync_copy(z_hbm.at[pl.ds(off, 640)], acc_sh.at[pl.ds(off, 640)])

        @pl.when((sid == 15) & (cid == 0))
        def _():
            pltpu.sync_copy(z_hbm.at[pl.ds(9600, 400)], acc_sh.at[pl.ds(9600, 400)])

        @pl.when((sid < 15) & (cid == 1))
        def _():
            pltpu.sync_copy(zeros2_hbm.at[pl.ds(off, 640)],
                            acc_sh.at[pl.ds(off, 640)])

        @pl.when((sid == 15) & (cid == 1))
        def _():
            pltpu.sync_copy(zeros2_hbm.at[pl.ds(9600, 400)],
                            acc_sh.at[pl.ds(9600, 400)])

        plsc.subcore_barrier()

        def chunk(i, carry):
            g = lax.div(i, NK)
            k = lax.rem(i, NK)
            gb = lax.rem(g, 2)
            b = lax.rem(i, NB)

            @pl.when((k == NB - LK) & (g + 1 < NG))
            def _():
                pltpu.async_copy(row4_hbm.at[wid, g + 1], ridx.at[1 - gb], isem)
                pltpu.async_copy(col4_hbm.at[wid, g + 1], cidx.at[1 - gb], isem)

            @pl.when((k == NK - LK) & (g + 1 < NG))
            def _():
                pltpu.make_async_copy(row4_hbm.at[wid, g + 1],
                                      ridx.at[1 - gb], isem).wait()
                pltpu.make_async_copy(col4_hbm.at[wid, g + 1],
                                      cidx.at[1 - gb], isem).wait()

            pltpu.make_async_copy(z_hbm.at[ridx.at[gb, k]],
                                  rows_v.at[b], gsem.at[b]).wait()
            pltpu.async_copy(rows_v.at[b], acc_sh.at[cidx.at[gb, k]],
                             ssem.at[b], add=True)

            j = i + LK
            bj = lax.rem(j, NB)
            gj = lax.div(j, NK)
            kj = lax.rem(j, NK)
            gbj = lax.rem(gj, 2)

            @pl.when((i >= NB - LK) & (j < NCHUNK))
            def _():
                pltpu.make_async_copy(rows_v.at[bj],
                                      acc_sh.at[cidx.at[0, 0]],
                                      ssem.at[bj]).wait()

            @pl.when(j < NCHUNK)
            def _():
                pltpu.async_copy(z_hbm.at[ridx.at[gbj, kj]],
                                 rows_v.at[bj], gsem.at[bj])
            return carry

        lax.fori_loop(0, NCHUNK, chunk, 0)
        for b in range(NB):
            pltpu.make_async_copy(rows_v.at[b], acc_sh.at[cidx.at[0, 0]],
                                  ssem.at[b]).wait()
        plsc.subcore_barrier()

        @pl.when(sid < 15)
        def _():
            pltpu.sync_copy(acc_sh.at[pl.ds(off, 640)],
                            out_hbm.at[cid].at[pl.ds(off, 640)])

        @pl.when(sid == 15)
        def _():
            pltpu.sync_copy(acc_sh.at[pl.ds(9600, 400)],
                            out_hbm.at[cid].at[pl.ds(9600, 400)])

    BN = 10000

    def tc1_body(d0, d1, x, w, b, dis_ref, z_ref):
        dis_b = 1.0 / jnp.sqrt(d0[...] + d1[...] + 1.0)
        dis_ref[...] = dis_b
        y = lax.dot_general(x[...], w[...], (((1,), (1,)), ((), ())),
                            preferred_element_type=jnp.float32)
        z_ref[...] = dis_b * (y + b[...])

    def tc1(d0, d1, x, w, b):
        return pl.pallas_call(
            tc1_body,
            grid=(N // BN,),
            in_specs=[
                pl.BlockSpec((BN, 1), lambda i: (i, 0)),
                pl.BlockSpec((BN, 1), lambda i: (i, 0)),
                pl.BlockSpec((BN, D), lambda i: (i, 0)),
                pl.BlockSpec((D, D), lambda i: (0, 0)),
                pl.BlockSpec((1, D), lambda i: (0, 0)),
            ],
            out_specs=[
                pl.BlockSpec((BN, 1), lambda i: (i, 0)),
                pl.BlockSpec((BN, D), lambda i: (i, 0)),
            ],
            out_shape=[
                jax.ShapeDtypeStruct((N, 1), jnp.float32),
                jax.ShapeDtypeStruct((N, D), jnp.float32),
            ],
        )(d0, d1, x, w, b)

    def tc2_body(p0, p1, dis, w, b, z_ref):
        dis_b = dis[...]
        h = jnp.maximum(dis_b * (p0[...] + p1[...]), 0.0)
        y = lax.dot_general(h, w[...], (((1,), (1,)), ((), ())),
                            preferred_element_type=jnp.float32)
        z_ref[...] = dis_b * (y + b[...])

    def tc2(p0, p1, dis, w, b):
        return pl.pallas_call(
            tc2_body,
            grid=(N // BN,),
            in_specs=[
                pl.BlockSpec((BN, D), lambda i: (i, 0)),
                pl.BlockSpec((BN, D), lambda i: (i, 0)),
                pl.BlockSpec((BN, 1), lambda i: (i, 0)),
                pl.BlockSpec((D, D), lambda i: (0, 0)),
                pl.BlockSpec((1, D), lambda i: (0, 0)),
            ],
            out_specs=pl.BlockSpec((BN, D), lambda i: (i, 0)),
            out_shape=jax.ShapeDtypeStruct((N, D), jnp.float32),
        )(p0, p1, dis, w, b)

    def tc3_body(q0, q1, dis, out_ref):
        out_ref[...] = jnp.maximum(dis[...] * (q0[...] + q1[...]), 0.0)

    def tc3(q0, q1, dis):
        return pl.pallas_call(
            tc3_body,
            grid=(N // BN,),
            in_specs=[
                pl.BlockSpec((BN, D), lambda i: (i, 0)),
                pl.BlockSpec((BN, D), lambda i: (i, 0)),
                pl.BlockSpec((BN, 1), lambda i: (i, 0)),
            ],
            out_specs=pl.BlockSpec((BN, D), lambda i: (i, 0)),
            out_shape=jax.ShapeDtypeStruct((N, D), jnp.float32),
        )(q0, q1, dis)

    return deg_kernel, edge_kernel, tc1, tc2, tc3


def kernel(x, edge_index, W1, b1, W2, b2):
    N, D = x.shape
    E = edge_index.shape[1]
    deg_kernel, edge_kernel, tc1, tc2, tc3 = _build(N, E, D)

    row4 = edge_index[0].reshape(NW, E // (NW * C * NK), NK, C)
    col4 = edge_index[1].reshape(NW, E // (NW * C * NK), NK, C)
    col3 = edge_index[1].reshape(NW, E // (NW * C), C)
    zeros1 = jnp.zeros((N,), jnp.float32)
    zeros2 = jnp.zeros((N, D), jnp.float32)

    degp = deg_kernel(col3, zeros1)
    d0 = degp[0, 0].reshape(N, 1)
    d1 = degp[1, 0].reshape(N, 1)
    dis, z1 = tc1(d0, d1, x, W1.reshape(D, D), b1.reshape(1, D))
    p = edge_kernel(z1, row4, col4, zeros2)
    z2 = tc2(p[0], p[1], dis, W2, b2.reshape(1, D))
    q = edge_kernel(z2, row4, col4, zeros2)
    return tc3(q[0], q[1], dis)


# speedup vs baseline: 1.0176x; 1.0176x over previous
import functools
import jax
import jax.numpy as jnp
from jax import lax
from jax.experimental import pallas as pl
from jax.experimental.pallas import tpu as pltpu
from jax.experimental.pallas import tpu_sc as plsc

NC = 2
NS = 16
NW = NC * NS
C = 80
NK = 5
NB = 4
LK = 3


@functools.lru_cache(maxsize=None)
def _build(N, E, D):
    EW = E // NW
    NCHUNK = EW // C
    assert E % NW == 0 and EW % C == 0

    mesh = plsc.VectorSubcoreMesh(core_axis_name="c", subcore_axis_name="s")

    @functools.partial(
        pl.kernel,
        out_type=jax.ShapeDtypeStruct((NC, 1, N), jnp.float32),
        mesh=mesh,
        scratch_types=[
            pltpu.VMEM((NCHUNK, C), jnp.int32),
            pltpu.VMEM((C,), jnp.float32),
            pltpu.VMEM_SHARED((N,), jnp.float32),
            pltpu.SemaphoreType.DMA,
        ],
    )
    def deg_kernel(col3_hbm, zeros1_hbm, out_hbm, cidx, ones_v, acc_sh, sem):
        cid = lax.axis_index("c")
        sid = lax.axis_index("s")
        wid = sid * NC + cid
        for j in range(C // 16):
            ones_v[pl.ds(16 * j, 16)] = jnp.full((16,), 1.0, jnp.float32)

        @pl.when(sid == 0)
        def _():
            pltpu.sync_copy(zeros1_hbm, acc_sh)

        pltpu.sync_copy(col3_hbm.at[wid], cidx)
        plsc.subcore_barrier()

        def chunk(i, carry):
            @pl.when(i >= 8)
            def _():
                pltpu.make_async_copy(ones_v, acc_sh.at[cidx.at[0]], sem).wait()

            pltpu.async_copy(ones_v, acc_sh.at[cidx.at[i]], sem, add=True)
            return carry

        lax.fori_loop(0, NCHUNK, chunk, 0)
        for _ in range(8):
            pltpu.make_async_copy(ones_v, acc_sh.at[cidx.at[0]], sem).wait()
        plsc.subcore_barrier()

        @pl.when(sid == 0)
        def _():
            pltpu.sync_copy(acc_sh, out_hbm.at[cid, 0])

    NG = NCHUNK // NK
    assert NCHUNK % NK == 0

    @functools.partial(
        pl.kernel,
        out_type=jax.ShapeDtypeStruct((NC, N, D), jnp.float32),
        mesh=mesh,
        scratch_types=[
            pltpu.VMEM((2, NK, C), jnp.int32),
            pltpu.VMEM((2, NK, C), jnp.int32),
            pltpu.VMEM((NB, C, D), jnp.float32),
            pltpu.VMEM_SHARED((N, D), jnp.float32),
            pltpu.SemaphoreType.DMA((NB,)),
            pltpu.SemaphoreType.DMA((NB,)),
            pltpu.SemaphoreType.DMA,
        ],
    )
    def edge_kernel(z_hbm, row4_hbm, col4_hbm, zeros2_hbm, out_hbm,
                    ridx, cidx, rows_v, acc_sh, gsem, ssem, isem):
        cid = lax.axis_index("c")
        sid = lax.axis_index("s")
        wid = sid * NC + cid

        pltpu.sync_copy(row4_hbm.at[wid, 0], ridx.at[0])
        pltpu.sync_copy(col4_hbm.at[wid, 0], cidx.at[0])
        for b in range(LK):
            pltpu.async_copy(z_hbm.at[ridx.at[0, b]], rows_v.at[b], gsem.at[b])

        off = pl.multiple_of(sid * 640, 8)

        @pl.when((sid < 15) & (cid == 0))
        def _():
            pltpu.sync_copy(z_hbm.at[pl.ds(off, 640)], acc_sh.at[pl.ds(off, 640)])

        @pl.when((sid == 15) & (cid == 0))
        def _():
            pltpu.sync_copy(z_hbm.at[pl.ds(9600, 400)], acc_sh.at[pl.ds(9600, 400)])

        @pl.when((sid < 15) & (cid == 1))
        def _():
            pltpu.sync_copy(zeros2_hbm.at[pl.ds(off, 640)],
                            acc_sh.at[pl.ds(off, 640)])

        @pl.when((sid == 15) & (cid == 1))
        def _():
            pltpu.sync_copy(zeros2_hbm.at[pl.ds(9600, 400)],
                            acc_sh.at[pl.ds(9600, 400)])

        plsc.subcore_barrier()

        def chunk(i, carry):
            g = lax.div(i, NK)
            k = lax.rem(i, NK)
            gb = lax.rem(g, 2)
            b = lax.rem(i, NB)

            @pl.when((k == NB - LK) & (g + 1 < NG))
            def _():
                pltpu.async_copy(row4_hbm.at[wid, g + 1], ridx.at[1 - gb], isem)
                pltpu.async_copy(col4_hbm.at[wid, g + 1], cidx.at[1 - gb], isem)

            @pl.when((k == NK - LK) & (g + 1 < NG))
            def _():
                pltpu.make_async_copy(row4_hbm.at[wid, g + 1],
                                      ridx.at[1 - gb], isem).wait()
                pltpu.make_async_copy(col4_hbm.at[wid, g + 1],
                                      cidx.at[1 - gb], isem).wait()

            pltpu.make_async_copy(z_hbm.at[ridx.at[gb, k]],
                                  rows_v.at[b], gsem.at[b]).wait()
            pltpu.async_copy(rows_v.at[b], acc_sh.at[cidx.at[gb, k]],
                             ssem.at[b], add=True)

            j = i + LK
            bj = lax.rem(j, NB)
            gj = lax.div(j, NK)
            kj = lax.rem(j, NK)
            gbj = lax.rem(gj, 2)

            @pl.when((i >= NB - LK) & (j < NCHUNK))
            def _():
                pltpu.make_async_copy(rows_v.at[bj],
                                      acc_sh.at[cidx.at[0, 0]],
                                      ssem.at[bj]).wait()

            @pl.when(j < NCHUNK)
            def _():
                pltpu.async_copy(z_hbm.at[ridx.at[gbj, kj]],
                                 rows_v.at[bj], gsem.at[bj])
            return carry

        lax.fori_loop(0, NCHUNK, chunk, 0)
        for b in range(NB):
            pltpu.make_async_copy(rows_v.at[b], acc_sh.at[cidx.at[0, 0]],
                                  ssem.at[b]).wait()
        plsc.subcore_barrier()

        @pl.when(sid < 15)
        def _():
            pltpu.sync_copy(acc_sh.at[pl.ds(off, 640)],
                            out_hbm.at[cid].at[pl.ds(off, 640)])

        @pl.when(sid == 15)
        def _():
            pltpu.sync_copy(acc_sh.at[pl.ds(9600, 400)],
                            out_hbm.at[cid].at[pl.ds(9600, 400)])

    BN = 5000

    def tc1_body(d0, d1, x, w, b, dis_ref, z_ref):
        dis_b = 1.0 / jnp.sqrt(d0[...] + d1[...] + 1.0)
        dis_ref[...] = dis_b
        y = lax.dot_general(x[...], w[...], (((1,), (1,)), ((), ())),
                            preferred_element_type=jnp.float32)
        z_ref[...] = dis_b * (y + b[...])

    def tc1(d0, d1, x, w, b):
        return pl.pallas_call(
            tc1_body,
            grid=(N // BN,),
            in_specs=[
                pl.BlockSpec((BN, 1), lambda i: (i, 0)),
                pl.BlockSpec((BN, 1), lambda i: (i, 0)),
                pl.BlockSpec((BN, D), lambda i: (i, 0)),
                pl.BlockSpec((D, D), lambda i: (0, 0)),
                pl.BlockSpec((1, D), lambda i: (0, 0)),
            ],
            out_specs=[
                pl.BlockSpec((BN, 1), lambda i: (i, 0)),
                pl.BlockSpec((BN, D), lambda i: (i, 0)),
            ],
            out_shape=[
                jax.ShapeDtypeStruct((N, 1), jnp.float32),
                jax.ShapeDtypeStruct((N, D), jnp.float32),
            ],
        )(d0, d1, x, w, b)

    def tc2_body(p0, p1, dis, w, b, z_ref):
        dis_b = dis[...]
        h = jnp.maximum(dis_b * (p0[...] + p1[...]), 0.0)
        y = lax.dot_general(h, w[...], (((1,), (1,)), ((), ())),
                            preferred_element_type=jnp.float32)
        z_ref[...] = dis_b * (y + b[...])

    def tc2(p0, p1, dis, w, b):
        return pl.pallas_call(
            tc2_body,
            grid=(N // BN,),
            in_specs=[
                pl.BlockSpec((BN, D), lambda i: (i, 0)),
                pl.BlockSpec((BN, D), lambda i: (i, 0)),
                pl.BlockSpec((BN, 1), lambda i: (i, 0)),
                pl.BlockSpec((D, D), lambda i: (0, 0)),
                pl.BlockSpec((1, D), lambda i: (0, 0)),
            ],
            out_specs=pl.BlockSpec((BN, D), lambda i: (i, 0)),
            out_shape=jax.ShapeDtypeStruct((N, D), jnp.float32),
        )(p0, p1, dis, w, b)

    def tc3_body(q0, q1, dis, out_ref):
        out_ref[...] = jnp.maximum(dis[...] * (q0[...] + q1[...]), 0.0)

    def tc3(q0, q1, dis):
        return pl.pallas_call(
            tc3_body,
            grid=(N // BN,),
            in_specs=[
                pl.BlockSpec((BN, D), lambda i: (i, 0)),
                pl.BlockSpec((BN, D), lambda i: (i, 0)),
                pl.BlockSpec((BN, 1), lambda i: (i, 0)),
            ],
            out_specs=pl.BlockSpec((BN, D), lambda i: (i, 0)),
            out_shape=jax.ShapeDtypeStruct((N, D), jnp.float32),
        )(q0, q1, dis)

    return deg_kernel, edge_kernel, tc1, tc2, tc3


def kernel(x, edge_index, W1, b1, W2, b2):
    N, D = x.shape
    E = edge_index.shape[1]
    deg_kernel, edge_kernel, tc1, tc2, tc3 = _build(N, E, D)

    row4 = edge_index[0].reshape(NW, E // (NW * C * NK), NK, C)
    col4 = edge_index[1].reshape(NW, E // (NW * C * NK), NK, C)
    col3 = edge_index[1].reshape(NW, E // (NW * C), C)
    zeros1 = jnp.zeros((N,), jnp.float32)
    zeros2 = jnp.zeros((N, D), jnp.float32)

    degp = deg_kernel(col3, zeros1)
    d0 = degp[0, 0].reshape(N, 1)
    d1 = degp[1, 0].reshape(N, 1)
    dis, z1 = tc1(d0, d1, x, W1.reshape(D, D), b1.reshape(1, D))
    p = edge_kernel(z1, row4, col4, zeros2)
    z2 = tc2(p[0], p[1], dis, W2, b2.reshape(1, D))
    q = edge_kernel(z2, row4, col4, zeros2)
    return tc3(q[0], q[1], dis)
